# baseline (device time: 194272 ns/iter reference)
import jax
import jax.numpy as jnp
from jax import lax
from jax.experimental import pallas as pl
from jax.experimental.pallas import tpu as pltpu

N_DEV = 16
ROWS = 2048
COLS = 2048
CHUNK = ROWS // N_DEV
HALF = CHUNK // 2
NSUB = 8
SUBC = COLS // NSUB


def kernel(partial, resid, gamma):
    partial2d = partial.reshape(ROWS, COLS)
    gamma2d = gamma.reshape(1, COLS)

    def body(
        partial_ref,
        resid_ref,
        gamma_ref,
        out_ref,
        comm_r,
        comm_l,
        part_r,
        part_l,
        resid_vmem,
        part_r_sems,
        part_l_sems,
        rs_send_r,
        rs_send_l,
        rs_recv_r,
        rs_recv_l,
        ag_send_r,
        ag_send_l,
        ag_recv_r,
        ag_recv_l,
        resid_sem,
    ):
        p = lax.axis_index("i")
        left = jnp.mod(p - 1 + N_DEV, N_DEV)
        right = jnp.mod(p + 1, N_DEV)

        barrier_sem = pltpu.get_barrier_semaphore()
        for nbr in (left, right):
            pl.semaphore_signal(
                barrier_sem,
                inc=1,
                device_id=(nbr,),
                device_id_type=pl.DeviceIdType.MESH,
            )
        pl.semaphore_wait(barrier_sem, 2)

        own_r_rows = jnp.mod(p + 1, N_DEV) * CHUNK
        own_l_rows = jnp.mod(p - 1 + N_DEV, N_DEV) * CHUNK + HALF

        resid_copies = [
            pltpu.make_async_copy(
                resid_ref.at[pl.ds(rows, HALF), :],
                resid_vmem.at[i],
                resid_sem.at[i],
            )
            for i, rows in enumerate((own_r_rows, own_l_rows))
        ]
        for c in resid_copies:
            c.start()

        def part_copy_r(h):
            rows = jnp.mod(p - h + N_DEV, N_DEV) * CHUNK
            return pltpu.make_async_copy(
                partial_ref.at[pl.ds(rows, HALF), :],
                part_r.at[h % 2],
                part_r_sems.at[h % 2],
            )

        def part_copy_l(h):
            rows = jnp.mod(p + h, N_DEV) * CHUNK + HALF
            return pltpu.make_async_copy(
                partial_ref.at[pl.ds(rows, HALF), :],
                part_l.at[h % 2],
                part_l_sems.at[h % 2],
            )

        copies_r = [part_copy_r(0)]
        copies_l = [part_copy_l(0)]
        copies_r[0].start()
        copies_l[0].start()

        def rs_rdma(ring, comm, src, h, s, sends, recvs, dev):
            del ring
            return pltpu.make_async_remote_copy(
                src_ref=src.at[:, pl.ds(s * SUBC, SUBC)],
                dst_ref=comm.at[h, :, pl.ds(s * SUBC, SUBC)],
                send_sem=sends.at[h % 2, s],
                recv_sem=recvs.at[h, s],
                device_id=(dev,),
                device_id_type=pl.DeviceIdType.MESH,
            )

        rs_r = {}
        rs_l = {}
        for h in range(N_DEV - 1):
            nxt_r = part_copy_r(h + 1)
            nxt_l = part_copy_l(h + 1)
            copies_r.append(nxt_r)
            copies_l.append(nxt_l)
            copies_r[h].wait()
            copies_l[h].wait()
            if h == 1:
                for s in range(NSUB):
                    rs_r[(0, s)].wait_send()
                    rs_l[(0, s)].wait_send()
            nxt_r.start()
            nxt_l.start()
            for s in range(NSUB):
                if h == 0:
                    src_r = part_r.at[0]
                    src_l = part_l.at[0]
                else:
                    cols = pl.ds(s * SUBC, SUBC)
                    rs_r[(h - 1, s)].wait_recv()
                    comm_r[h - 1, :, cols] = (
                        comm_r[h - 1, :, cols] + part_r[h % 2, :, cols]
                    )
                    rs_l[(h - 1, s)].wait_recv()
                    comm_l[h - 1, :, cols] = (
                        comm_l[h - 1, :, cols] + part_l[h % 2, :, cols]
                    )
                    src_r = comm_r.at[h - 1]
                    src_l = comm_l.at[h - 1]
                rdma_r = rs_rdma("r", comm_r, src_r, h, s, rs_send_r, rs_recv_r, right)
                rdma_l = rs_rdma("l", comm_l, src_l, h, s, rs_send_l, rs_recv_l, left)
                if h >= 3:
                    rs_r[(h - 2, s)].wait_send()
                    rs_l[(h - 2, s)].wait_send()
                rdma_r.start()
                rdma_l.start()
                rs_r[(h, s)] = rdma_r
                rs_l[(h, s)] = rdma_l

        copies_r[N_DEV - 1].wait()
        copies_l[N_DEV - 1].wait()
        for s in range(NSUB):
            rs_r[(N_DEV - 2, s)].wait_recv()
            rs_l[(N_DEV - 2, s)].wait_recv()
        slot = (N_DEV - 1) % 2
        red_r = comm_r[N_DEV - 2, :, :] + part_r[slot, :, :]
        red_l = comm_l[N_DEV - 2, :, :] + part_l[slot, :, :]

        g = gamma_ref[0, :][None, :]
        for red, rv, rows in (
            (red_r, 0, own_r_rows),
            (red_l, 1, own_l_rows),
        ):
            resid_copies[rv].wait()
            y = red + resid_vmem[rv, :, :]
            rms = jnp.sqrt(jnp.mean(y * y, axis=-1, keepdims=True) + 1e-6)
            out_ref[pl.ds(rows, HALF), :] = y / rms * g

        for s in range(NSUB):
            for rs in (rs_r, rs_l):
                rs[(N_DEV - 3, s)].wait_send()
                rs[(N_DEV - 2, s)].wait_send()

        def ag_rdma(rows, s, send_sems, s_idx, recv_sems, r_idx, dev):
            return pltpu.make_async_remote_copy(
                src_ref=out_ref.at[pl.ds(rows, HALF), pl.ds(s * SUBC, SUBC)],
                dst_ref=out_ref.at[pl.ds(rows, HALF), pl.ds(s * SUBC, SUBC)],
                send_sem=send_sems.at[s_idx, s],
                recv_sem=recv_sems.at[r_idx, s],
                device_id=(dev,),
                device_id_type=pl.DeviceIdType.MESH,
            )

        ag_r = {}
        ag_l = {}
        for h in range(N_DEV - 1):
            r_rows = jnp.mod(p + 1 - h + N_DEV, N_DEV) * CHUNK
            l_rows = jnp.mod(p - 1 + h, N_DEV) * CHUNK + HALF
            for s in range(NSUB):
                if h >= 1:
                    ag_rdma(r_rows, s, ag_send_r, (h - 1) % 2, ag_recv_r, h - 1, right).wait_recv()
                    ag_rdma(l_rows, s, ag_send_l, (h - 1) % 2, ag_recv_l, h - 1, left).wait_recv()
                if h >= 2:
                    ag_r[(h - 2, s)].wait_send()
                    ag_l[(h - 2, s)].wait_send()
                rdma_r = ag_rdma(r_rows, s, ag_send_r, h % 2, ag_recv_r, h, right)
                rdma_l = ag_rdma(l_rows, s, ag_send_l, h % 2, ag_recv_l, h, left)
                rdma_r.start()
                rdma_l.start()
                ag_r[(h, s)] = rdma_r
                ag_l[(h, s)] = rdma_l

        fr = jnp.mod(p + 2, N_DEV) * CHUNK
        fl = jnp.mod(p - 2 + N_DEV, N_DEV) * CHUNK + HALF
        for s in range(NSUB):
            ag_rdma(fr, s, ag_send_r, 0, ag_recv_r, N_DEV - 2, right).wait_recv()
            ag_rdma(fl, s, ag_send_l, 0, ag_recv_l, N_DEV - 2, left).wait_recv()

        for s in range(NSUB):
            for ag in (ag_r, ag_l):
                ag[(N_DEV - 3, s)].wait_send()
                ag[(N_DEV - 2, s)].wait_send()

    return pl.pallas_call(
        body,
        out_shape=jax.ShapeDtypeStruct((ROWS, COLS), jnp.float32),
        in_specs=[
            pl.BlockSpec(memory_space=pl.ANY),
            pl.BlockSpec(memory_space=pl.ANY),
            pl.BlockSpec(memory_space=pltpu.VMEM),
        ],
        out_specs=pl.BlockSpec(memory_space=pltpu.VMEM),
        scratch_shapes=[
            pltpu.VMEM((N_DEV - 1, HALF, COLS), jnp.float32),
            pltpu.VMEM((N_DEV - 1, HALF, COLS), jnp.float32),
            pltpu.VMEM((2, HALF, COLS), jnp.float32),
            pltpu.VMEM((2, HALF, COLS), jnp.float32),
            pltpu.VMEM((2, HALF, COLS), jnp.float32),
            pltpu.SemaphoreType.DMA((2,)),
            pltpu.SemaphoreType.DMA((2,)),
            pltpu.SemaphoreType.DMA((2, NSUB)),
            pltpu.SemaphoreType.DMA((2, NSUB)),
            pltpu.SemaphoreType.DMA((N_DEV - 1, NSUB)),
            pltpu.SemaphoreType.DMA((N_DEV - 1, NSUB)),
            pltpu.SemaphoreType.DMA((2, NSUB)),
            pltpu.SemaphoreType.DMA((2, NSUB)),
            pltpu.SemaphoreType.DMA((N_DEV - 1, NSUB)),
            pltpu.SemaphoreType.DMA((N_DEV - 1, NSUB)),
            pltpu.SemaphoreType.DMA((2,)),
        ],
        compiler_params=pltpu.CompilerParams(collective_id=0),
    )(partial2d, resid, gamma2d)


# device time: 188314 ns/iter; 1.0316x vs baseline; 1.0316x over previous
import jax
import jax.numpy as jnp
from jax import lax
from jax.experimental import pallas as pl
from jax.experimental.pallas import tpu as pltpu

N_DEV = 16
ROWS = 2048
COLS = 2048
CHUNK = ROWS // N_DEV
HALF = CHUNK // 2
NSUB = 4
SUBC = COLS // NSUB


def kernel(partial, resid, gamma):
    partial2d = partial.reshape(ROWS, COLS)
    gamma2d = gamma.reshape(1, COLS)

    def body(
        partial_ref,
        resid_ref,
        gamma_ref,
        out_ref,
        comm_r,
        comm_l,
        part_r,
        part_l,
        resid_vmem,
        part_r_sems,
        part_l_sems,
        rs_send_r,
        rs_send_l,
        rs_recv_r,
        rs_recv_l,
        ag_send_r,
        ag_send_l,
        ag_recv_r,
        ag_recv_l,
        resid_sem,
    ):
        p = lax.axis_index("i")
        left = jnp.mod(p - 1 + N_DEV, N_DEV)
        right = jnp.mod(p + 1, N_DEV)

        barrier_sem = pltpu.get_barrier_semaphore()
        for nbr in (left, right):
            pl.semaphore_signal(
                barrier_sem,
                inc=1,
                device_id=(nbr,),
                device_id_type=pl.DeviceIdType.MESH,
            )
        pl.semaphore_wait(barrier_sem, 2)

        own_r_rows = jnp.mod(p + 1, N_DEV) * CHUNK
        own_l_rows = jnp.mod(p - 1 + N_DEV, N_DEV) * CHUNK + HALF

        resid_copies = [
            pltpu.make_async_copy(
                resid_ref.at[pl.ds(rows, HALF), :],
                resid_vmem.at[i],
                resid_sem.at[i],
            )
            for i, rows in enumerate((own_r_rows, own_l_rows))
        ]
        for c in resid_copies:
            c.start()

        def part_copy_r(h):
            rows = jnp.mod(p - h + N_DEV, N_DEV) * CHUNK
            return pltpu.make_async_copy(
                partial_ref.at[pl.ds(rows, HALF), :],
                part_r.at[h % 2],
                part_r_sems.at[h % 2],
            )

        def part_copy_l(h):
            rows = jnp.mod(p + h, N_DEV) * CHUNK + HALF
            return pltpu.make_async_copy(
                partial_ref.at[pl.ds(rows, HALF), :],
                part_l.at[h % 2],
                part_l_sems.at[h % 2],
            )

        copies_r = [part_copy_r(0)]
        copies_l = [part_copy_l(0)]
        copies_r[0].start()
        copies_l[0].start()

        def rs_rdma(ring, comm, src, h, s, sends, recvs, dev):
            del ring
            return pltpu.make_async_remote_copy(
                src_ref=src.at[:, pl.ds(s * SUBC, SUBC)],
                dst_ref=comm.at[h, :, pl.ds(s * SUBC, SUBC)],
                send_sem=sends.at[h % 2, s],
                recv_sem=recvs.at[h, s],
                device_id=(dev,),
                device_id_type=pl.DeviceIdType.MESH,
            )

        rs_r = {}
        rs_l = {}
        for h in range(N_DEV - 1):
            nxt_r = part_copy_r(h + 1)
            nxt_l = part_copy_l(h + 1)
            copies_r.append(nxt_r)
            copies_l.append(nxt_l)
            copies_r[h].wait()
            copies_l[h].wait()
            if h == 1:
                for s in range(NSUB):
                    rs_r[(0, s)].wait_send()
                    rs_l[(0, s)].wait_send()
            nxt_r.start()
            nxt_l.start()
            for s in range(NSUB):
                if h == 0:
                    src_r = part_r.at[0]
                    src_l = part_l.at[0]
                else:
                    cols = pl.ds(s * SUBC, SUBC)
                    rs_r[(h - 1, s)].wait_recv()
                    comm_r[h - 1, :, cols] = (
                        comm_r[h - 1, :, cols] + part_r[h % 2, :, cols]
                    )
                    rs_l[(h - 1, s)].wait_recv()
                    comm_l[h - 1, :, cols] = (
                        comm_l[h - 1, :, cols] + part_l[h % 2, :, cols]
                    )
                    src_r = comm_r.at[h - 1]
                    src_l = comm_l.at[h - 1]
                rdma_r = rs_rdma("r", comm_r, src_r, h, s, rs_send_r, rs_recv_r, right)
                rdma_l = rs_rdma("l", comm_l, src_l, h, s, rs_send_l, rs_recv_l, left)
                if h >= 3:
                    rs_r[(h - 2, s)].wait_send()
                    rs_l[(h - 2, s)].wait_send()
                rdma_r.start()
                rdma_l.start()
                rs_r[(h, s)] = rdma_r
                rs_l[(h, s)] = rdma_l

        def ag_rdma(rows, s, send_sems, s_idx, recv_sems, r_idx, dev):
            return pltpu.make_async_remote_copy(
                src_ref=out_ref.at[pl.ds(rows, HALF), pl.ds(s * SUBC, SUBC)],
                dst_ref=out_ref.at[pl.ds(rows, HALF), pl.ds(s * SUBC, SUBC)],
                send_sem=send_sems.at[s_idx, s],
                recv_sem=recv_sems.at[r_idx, s],
                device_id=(dev,),
                device_id_type=pl.DeviceIdType.MESH,
            )

        copies_r[N_DEV - 1].wait()
        copies_l[N_DEV - 1].wait()
        resid_copies[0].wait()
        resid_copies[1].wait()
        slot = (N_DEV - 1) % 2
        ys_r = []
        ys_l = []
        ssq_r = []
        ssq_l = []
        for s in range(NSUB):
            cols = pl.ds(s * SUBC, SUBC)
            rs_r[(N_DEV - 2, s)].wait_recv()
            y = (
                comm_r[N_DEV - 2, :, cols]
                + part_r[slot, :, cols]
                + resid_vmem[0, :, cols]
            )
            ys_r.append(y)
            ssq_r.append(jnp.sum(y * y, axis=-1, keepdims=True))
            rs_l[(N_DEV - 2, s)].wait_recv()
            y = (
                comm_l[N_DEV - 2, :, cols]
                + part_l[slot, :, cols]
                + resid_vmem[1, :, cols]
            )
            ys_l.append(y)
            ssq_l.append(jnp.sum(y * y, axis=-1, keepdims=True))

        inv_r = lax.rsqrt(sum(ssq_r) / COLS + 1e-6)
        inv_l = lax.rsqrt(sum(ssq_l) / COLS + 1e-6)
        ag_r = {}
        ag_l = {}
        for s in range(NSUB):
            cols = pl.ds(s * SUBC, SUBC)
            g_s = gamma_ref[0, s * SUBC : (s + 1) * SUBC][None, :]
            out_ref[pl.ds(own_r_rows, HALF), cols] = ys_r[s] * inv_r * g_s
            rdma = ag_rdma(own_r_rows, s, ag_send_r, 0, ag_recv_r, 0, right)
            rdma.start()
            ag_r[(0, s)] = rdma
            out_ref[pl.ds(own_l_rows, HALF), cols] = ys_l[s] * inv_l * g_s
            rdma = ag_rdma(own_l_rows, s, ag_send_l, 0, ag_recv_l, 0, left)
            rdma.start()
            ag_l[(0, s)] = rdma

        for s in range(NSUB):
            for rs in (rs_r, rs_l):
                rs[(N_DEV - 3, s)].wait_send()
                rs[(N_DEV - 2, s)].wait_send()

        for h in range(1, N_DEV - 1):
            r_rows = jnp.mod(p + 1 - h + N_DEV, N_DEV) * CHUNK
            l_rows = jnp.mod(p - 1 + h, N_DEV) * CHUNK + HALF
            for s in range(NSUB):
                if h >= 1:
                    ag_rdma(r_rows, s, ag_send_r, (h - 1) % 2, ag_recv_r, h - 1, right).wait_recv()
                    ag_rdma(l_rows, s, ag_send_l, (h - 1) % 2, ag_recv_l, h - 1, left).wait_recv()
                if h >= 2:
                    ag_r[(h - 2, s)].wait_send()
                    ag_l[(h - 2, s)].wait_send()
                rdma_r = ag_rdma(r_rows, s, ag_send_r, h % 2, ag_recv_r, h, right)
                rdma_l = ag_rdma(l_rows, s, ag_send_l, h % 2, ag_recv_l, h, left)
                rdma_r.start()
                rdma_l.start()
                ag_r[(h, s)] = rdma_r
                ag_l[(h, s)] = rdma_l

        fr = jnp.mod(p + 2, N_DEV) * CHUNK
        fl = jnp.mod(p - 2 + N_DEV, N_DEV) * CHUNK + HALF
        for s in range(NSUB):
            ag_rdma(fr, s, ag_send_r, 0, ag_recv_r, N_DEV - 2, right).wait_recv()
            ag_rdma(fl, s, ag_send_l, 0, ag_recv_l, N_DEV - 2, left).wait_recv()

        for s in range(NSUB):
            for ag in (ag_r, ag_l):
                ag[(N_DEV - 3, s)].wait_send()
                ag[(N_DEV - 2, s)].wait_send()

    return pl.pallas_call(
        body,
        out_shape=jax.ShapeDtypeStruct((ROWS, COLS), jnp.float32),
        in_specs=[
            pl.BlockSpec(memory_space=pl.ANY),
            pl.BlockSpec(memory_space=pl.ANY),
            pl.BlockSpec(memory_space=pltpu.VMEM),
        ],
        out_specs=pl.BlockSpec(memory_space=pltpu.VMEM),
        scratch_shapes=[
            pltpu.VMEM((N_DEV - 1, HALF, COLS), jnp.float32),
            pltpu.VMEM((N_DEV - 1, HALF, COLS), jnp.float32),
            pltpu.VMEM((2, HALF, COLS), jnp.float32),
            pltpu.VMEM((2, HALF, COLS), jnp.float32),
            pltpu.VMEM((2, HALF, COLS), jnp.float32),
            pltpu.SemaphoreType.DMA((2,)),
            pltpu.SemaphoreType.DMA((2,)),
            pltpu.SemaphoreType.DMA((2, NSUB)),
            pltpu.SemaphoreType.DMA((2, NSUB)),
            pltpu.SemaphoreType.DMA((N_DEV - 1, NSUB)),
            pltpu.SemaphoreType.DMA((N_DEV - 1, NSUB)),
            pltpu.SemaphoreType.DMA((2, NSUB)),
            pltpu.SemaphoreType.DMA((2, NSUB)),
            pltpu.SemaphoreType.DMA((N_DEV - 1, NSUB)),
            pltpu.SemaphoreType.DMA((N_DEV - 1, NSUB)),
            pltpu.SemaphoreType.DMA((2,)),
        ],
        compiler_params=pltpu.CompilerParams(collective_id=0),
    )(partial2d, resid, gamma2d)
